# baseline (speedup 1.0000x reference)
import functools

import jax
import jax.numpy as jnp
from jax import lax
from jax.experimental import pallas as pl
from jax.experimental.pallas import tpu as pltpu
from jax.experimental.pallas import tpu_sc as plsc

NC = 2
NS = 16
NW = NC * NS
L = 16


def _build(M, K, NF, NP):
    rpw = M // NW
    R = 64
    C = rpw // R
    NG = NP // L
    PW = R * K

    mesh = plsc.VectorSubcoreMesh(core_axis_name="c", subcore_axis_name="s")

    @functools.partial(
        pl.kernel,
        out_type=jax.ShapeDtypeStruct((M, NF), jnp.float32),
        mesh=mesh,
        scratch_types=[
            pltpu.VMEM((NP,), jnp.int32),
            pltpu.VMEM((NP,), jnp.int32),
            pltpu.VMEM((PW,), jnp.float32),
            pltpu.VMEM((PW,), jnp.float32),
            pltpu.VMEM((R, NF), jnp.float32),
            pltpu.VMEM((R, NF), jnp.float32),
            pltpu.SemaphoreType.DMA,
            pltpu.SemaphoreType.DMA,
            pltpu.SemaphoreType.DMA,
            pltpu.SemaphoreType.DMA,
        ],
        compiler_params=pltpu.CompilerParams(
            use_tc_tiling_on_sc=True,
            needs_layout_passes=False,
            disable_bounds_checks=True,
        ),
    )
    def k(x_hbm, idx_hbm, out_hbm, idxv, colt, xpa, xpb, outa, outb,
          isa, isb, osa, osb):
        xps, outs = [xpa, xpb], [outa, outb]
        isems, osems = [isa, isb], [osa, osb]
        wid = lax.axis_index("s") * NC + lax.axis_index("c")
        row0 = wid * rpw

        def prefetch_first(n, b):
            return pltpu.async_copy(
                x_hbm.at[pl.ds((row0 + n * R) * K, PW)], xps[b], isems[b]
            )

        prefetch_first(0, 0)

        pltpu.sync_copy(idx_hbm, idxv)
        for g in range(NG):
            v = idxv[pl.ds(g * L, L)]
            colt[pl.ds(g * L, L)] = (v >> 7) * 1024 + (v & 127)

        def issue_in(n, b):
            return pltpu.async_copy(
                x_hbm.at[pl.ds((row0 + n * R) * K, PW)], xps[b], isems[b]
            )

        def wait_in(b):
            pltpu.make_async_copy(
                x_hbm.at[pl.ds(0, PW)], xps[b], isems[b]
            ).wait()

        def issue_out(ch, b):
            return pltpu.async_copy(
                outs[b], out_hbm.at[pl.ds(row0 + ch * R, R)], osems[b]
            )

        def wait_out(b):
            pltpu.make_async_copy(
                outs[b], out_hbm.at[pl.ds(0, R)], osems[b]
            ).wait()

        def compute_chunk(xp, outv):
            @plsc.parallel_loop(0, NG)
            def gbody(g):
                g16 = pl.multiple_of(g * L, L)
                colp = colt[pl.ds(g16, L)]
                off = jnp.minimum(g * L, NF - L)

                @plsc.parallel_loop(0, R // 8, carry=colp, unroll=4)
                def trbody(tr, gidx):
                    r0 = tr * 8
                    for s in range(8):
                        vals = plsc.load_gather(xp, [gidx + s * 128])
                        outv[r0 + s, pl.ds(off, L)] = vals
                    return gidx + 4096

        def citer(it, _):
            for cc in range(2):
                ch = 2 * it + cc
                wait_in(cc)

                @pl.when(ch + 1 < C)
                def _():
                    issue_in(ch + 1, cc ^ 1)

                @pl.when(ch >= 2)
                def _():
                    wait_out(cc)

                compute_chunk(xps[cc], outs[cc])
                issue_out(ch, cc)
            return 0

        lax.fori_loop(0, C // 2, citer, 0)
        wait_out(0)
        wait_out(1)

    return k


def kernel(x, feature_indices):
    M, K = x.shape
    NF = feature_indices.shape[0]
    G = NF // L
    rem = NF % L
    if rem:
        idx_pad = jnp.concatenate(
            [feature_indices[: G * L], feature_indices[NF - L :]]
        )
    else:
        idx_pad = feature_indices
    NP = idx_pad.shape[0]

    x1 = jnp.transpose(
        x.reshape(M // 8, 8, K // 128, 128), (0, 2, 1, 3)
    ).reshape(-1)

    k = _build(M, K, NF, NP)
    return k(x1, idx_pad.astype(jnp.int32))
